# Initial kernel scaffold; baseline (speedup 1.0000x reference)
import jax
import jax.numpy as jnp
from jax.experimental import pallas as pl


def kernel(x, edge_index, W1_0, b1_0, W2_0, b2_0, W1_1, b1_1, W2_1, b2_1, W1_2, b1_2, W2_2, b2_2):
    raise NotImplementedError("write your pallas kernel here")


# speedup vs baseline: 2.6680x; 2.6680x over previous
import functools

import jax
import jax.numpy as jnp
from jax import lax
from jax.experimental import pallas as pl
from jax.experimental.pallas import tpu as pltpu
from jax.experimental.pallas import tpu_sc as plsc

N = 10000
E = 160000
D = 256
HALF = D // 2

NC = 2
NS = 16
LANES = 128
GROUPS_PER_LOAD = 8
SUBS_PER_LOAD = 4
GROUPS_PER_SUB = GROUPS_PER_LOAD // SUBS_PER_LOAD
SUB = LANES * GROUPS_PER_SUB

E_PAD = ((E + NS * GROUPS_PER_LOAD * LANES - 1)
         // (NS * GROUPS_PER_LOAD * LANES)) * (NS * GROUPS_PER_LOAD * LANES)
NGRP = E_PAD // LANES
GRP_PER_TILE = NGRP // NS
LOADS_PER_TILE = GRP_PER_TILE // GROUPS_PER_LOAD

TRASH = N
ACC_ROWS = N + 8
ROWS_PER_TILE = 624
TAIL_ROWS = N - NS * ROWS_PER_TILE


def _sc_aggregate(xv, idx2, dstg, x):
    mesh = plsc.VectorSubcoreMesh(
        core_axis_name="c", subcore_axis_name="s", num_cores=NC,
        num_subcores=NS)

    @functools.partial(
        pl.kernel,
        mesh=mesh,
        out_type=jax.ShapeDtypeStruct((NC, N, HALF), jnp.float32),
        scratch_types=[
            pltpu.VMEM((GROUPS_PER_LOAD, LANES), jnp.int32),
            pltpu.VMEM((GROUPS_PER_LOAD, LANES), jnp.int32),
            pltpu.VMEM((SUB, HALF), jnp.float32),
            pltpu.VMEM_SHARED((ACC_ROWS, HALF), jnp.float32),
            pltpu.SemaphoreType.DMA,
        ],
    )
    def k(xv_hbm, idx2_hbm, dstg_hbm, x_hbm, out_hbm,
          src_v, dst_v, rows_v, acc, sem):
        c = lax.axis_index("c")
        s = lax.axis_index("s")

        r0 = s * ROWS_PER_TILE
        pltpu.sync_copy(
            x_hbm.at[pl.ds(r0, ROWS_PER_TILE), pl.ds(c * HALF, HALF)],
            acc.at[pl.ds(r0, ROWS_PER_TILE)])

        @pl.when(s == 0)
        def _():
            pltpu.sync_copy(
                x_hbm.at[pl.ds(NS * ROWS_PER_TILE, TAIL_ROWS),
                         pl.ds(c * HALF, HALF)],
                acc.at[pl.ds(NS * ROWS_PER_TILE, TAIL_ROWS)])

        plsc.subcore_barrier()

        def load_body(t, carry):
            g0 = s * GRP_PER_TILE + t * GROUPS_PER_LOAD
            pltpu.sync_copy(idx2_hbm.at[c, pl.ds(g0, GROUPS_PER_LOAD)],
                            src_v)
            pltpu.sync_copy(dstg_hbm.at[pl.ds(g0, GROUPS_PER_LOAD)], dst_v)
            for r in range(SUBS_PER_LOAD):
                copies = []
                for j in range(GROUPS_PER_SUB):
                    jj = r * GROUPS_PER_SUB + j
                    copies.append(pltpu.async_copy(
                        xv_hbm.at[src_v.at[jj]],
                        rows_v.at[pl.ds(j * LANES, LANES)], sem))
                for cp in copies:
                    cp.wait()
                for j in range(GROUPS_PER_SUB):
                    jj = r * GROUPS_PER_SUB + j
                    pltpu.sync_copy(rows_v.at[pl.ds(j * LANES, LANES)],
                                    acc.at[dst_v.at[jj]], add=True)
            return carry

        lax.fori_loop(0, LOADS_PER_TILE, load_body, 0)
        plsc.subcore_barrier()

        pltpu.sync_copy(acc.at[pl.ds(r0, ROWS_PER_TILE)],
                        out_hbm.at[c, pl.ds(r0, ROWS_PER_TILE)])

        @pl.when(s == 0)
        def _():
            pltpu.sync_copy(
                acc.at[pl.ds(NS * ROWS_PER_TILE, TAIL_ROWS)],
                out_hbm.at[c, pl.ds(NS * ROWS_PER_TILE, TAIL_ROWS)])

    return k(xv, idx2, dstg, x)


def _mlp_kern(agg_ref, w1_ref, b1_ref, w2_ref, b2_ref, out_ref, *, leaky):
    a = agg_ref[...]
    h0 = jnp.concatenate([a[0], a[1]], axis=1)
    h = jnp.maximum(
        jnp.dot(h0, w1_ref[...], preferred_element_type=jnp.float32)
        + b1_ref[...], 0.0)
    o = (jnp.dot(h, w2_ref[...], preferred_element_type=jnp.float32)
         + b2_ref[...])
    if leaky:
        o = jnp.where(o >= 0, o, 0.01 * o)
    out_ref[...] = o


def _tc_mlp(agg, w1, b1, w2, b2, leaky):
    BN = 1000
    grid = (N // BN,)
    return pl.pallas_call(
        functools.partial(_mlp_kern, leaky=leaky),
        grid=grid,
        in_specs=[
            pl.BlockSpec((NC, BN, HALF), lambda i: (0, i, 0)),
            pl.BlockSpec((D, D), lambda i: (0, 0)),
            pl.BlockSpec((1, D), lambda i: (0, 0)),
            pl.BlockSpec((D, D), lambda i: (0, 0)),
            pl.BlockSpec((1, D), lambda i: (0, 0)),
        ],
        out_specs=pl.BlockSpec((BN, D), lambda i: (i, 0)),
        out_shape=jax.ShapeDtypeStruct((N, D), jnp.float32),
    )(agg, w1, b1.reshape(1, D), w2, b2.reshape(1, D))


def kernel(x, edge_index, W1_0, b1_0, W2_0, b2_0, W1_1, b1_1, W2_1, b2_1,
           W1_2, b1_2, W2_2, b2_2):
    src = edge_index[0].astype(jnp.int32)
    dst = edge_index[1].astype(jnp.int32)
    pad = E_PAD - E
    src_p = jnp.concatenate([src, jnp.zeros((pad,), jnp.int32)])
    dst_p = jnp.concatenate([dst, jnp.full((pad,), TRASH, jnp.int32)])
    idx2 = jnp.stack([src_p * 2, src_p * 2 + 1]).reshape(NC, NGRP, LANES)
    dstg = dst_p.reshape(NGRP, LANES)

    h = x
    for (w1, b1, w2, b2, leaky) in (
            (W1_0, b1_0, W2_0, b2_0, True),
            (W1_1, b1_1, W2_1, b2_1, True),
            (W1_2, b1_2, W2_2, b2_2, False)):
        xplus = _sc_aggregate(h.reshape(2 * N, HALF), idx2, dstg, h)
        h = _tc_mlp(xplus, w1, b1, w2, b2, leaky)
    return h


# speedup vs baseline: 2.8873x; 1.0822x over previous
import functools

import jax
import jax.numpy as jnp
from jax import lax
from jax.experimental import pallas as pl
from jax.experimental.pallas import tpu as pltpu
from jax.experimental.pallas import tpu_sc as plsc

N = 10000
E = 160000
D = 256
HALF = D // 2

NC = 2
NS = 16
LANES = 128
GROUPS_PER_LOAD = 8

E_PAD = ((E + NS * GROUPS_PER_LOAD * LANES - 1)
         // (NS * GROUPS_PER_LOAD * LANES)) * (NS * GROUPS_PER_LOAD * LANES)
NGRP = E_PAD // LANES
NGRP_PAD = NGRP + GROUPS_PER_LOAD
GRP_PER_TILE = NGRP // NS
LOADS_PER_TILE = GRP_PER_TILE // GROUPS_PER_LOAD
DBL = LOADS_PER_TILE // 2

TRASH = N
ACC_ROWS = N + 8
ROWS_PER_TILE = 624
TAIL_ROWS = N - NS * ROWS_PER_TILE


def _sc_aggregate(xv, idx2, dstg, x):
    mesh = plsc.VectorSubcoreMesh(
        core_axis_name="c", subcore_axis_name="s", num_cores=NC,
        num_subcores=NS)

    @functools.partial(
        pl.kernel,
        mesh=mesh,
        out_type=jax.ShapeDtypeStruct((NC, N, HALF), jnp.float32),
        scratch_types=[
            pltpu.VMEM((2, 2, GROUPS_PER_LOAD, LANES), jnp.int32),
            pltpu.VMEM((2, LANES, HALF), jnp.float32),
            pltpu.VMEM_SHARED((ACC_ROWS, HALF), jnp.float32),
            pltpu.SemaphoreType.DMA,
            pltpu.SemaphoreType.DMA,
            pltpu.SemaphoreType.DMA,
            pltpu.SemaphoreType.DMA,
            pltpu.SemaphoreType.DMA,
        ],
    )
    def k(xv_hbm, idx2_hbm, dstg_hbm, x_hbm, out_hbm,
          idx_v, rows_v, acc, sg0, sg1, ss0, ss1, si):
        c = lax.axis_index("c")
        s = lax.axis_index("s")
        sem_g = (sg0, sg1)
        sem_s = (ss0, ss1)

        def g_start(slot, grp, buf):
            return pltpu.async_copy(xv_hbm.at[idx_v.at[slot, 0, grp]],
                                    rows_v.at[buf], sem_g[buf])

        def s_start(slot, grp, buf):
            return pltpu.async_copy(rows_v.at[buf],
                                    acc.at[idx_v.at[slot, 1, grp]],
                                    sem_s[buf], add=True)

        def g_wait_recon(buf):
            pltpu.make_async_copy(xv_hbm.at[pl.ds(0, LANES)],
                                  rows_v.at[buf], sem_g[buf]).wait()

        def s_wait_recon(buf):
            pltpu.make_async_copy(xv_hbm.at[pl.ds(0, LANES)],
                                  rows_v.at[buf], sem_s[buf]).wait()

        def idx_load(slot, blk):
            g0 = s * GRP_PER_TILE + blk * GROUPS_PER_LOAD
            d1 = pltpu.async_copy(idx2_hbm.at[c, pl.ds(g0, GROUPS_PER_LOAD)],
                                  idx_v.at[slot, 0], si)
            d2 = pltpu.async_copy(dstg_hbm.at[pl.ds(g0, GROUPS_PER_LOAD)],
                                  idx_v.at[slot, 1], si)
            return d1, d2

        r0 = s * ROWS_PER_TILE
        pltpu.sync_copy(
            x_hbm.at[pl.ds(r0, ROWS_PER_TILE), pl.ds(c * HALF, HALF)],
            acc.at[pl.ds(r0, ROWS_PER_TILE)])

        @pl.when(s == 0)
        def _():
            pltpu.sync_copy(
                x_hbm.at[pl.ds(NS * ROWS_PER_TILE, TAIL_ROWS),
                         pl.ds(c * HALF, HALF)],
                acc.at[pl.ds(NS * ROWS_PER_TILE, TAIL_ROWS)])

        plsc.subcore_barrier()

        p1, p2 = idx_load(0, 0)
        p1.wait()
        p2.wait()
        g_start(0, 0, 0)

        def body(t, carry):
            sc_d = [None, None]
            g_d = [None, None]
            idx_d = None
            for j in range(2 * GROUPS_PER_LOAD):
                buf = j % 2
                nbuf = (j + 1) % 2
                if j == 0:
                    @pl.when(t > 0)
                    def _():
                        s_wait_recon(1)
                else:
                    sc_d[nbuf].wait()
                if j == 0:
                    idx_d = idx_load(1, 2 * t + 1)
                if j == GROUPS_PER_LOAD:
                    idx_d = idx_load(0, 2 * t + 2)
                if j == GROUPS_PER_LOAD - 1 or j == 2 * GROUPS_PER_LOAD - 1:
                    idx_d[0].wait()
                    idx_d[1].wait()
                if j < GROUPS_PER_LOAD - 1:
                    ns, ng = 0, j + 1
                elif j < 2 * GROUPS_PER_LOAD - 1:
                    ns, ng = 1, (j + 1) % GROUPS_PER_LOAD
                else:
                    ns, ng = 0, 0
                g_d[nbuf] = g_start(ns, ng, nbuf)
                if j == 0:
                    g_wait_recon(0)
                else:
                    g_d[buf].wait()
                slot = 0 if j < GROUPS_PER_LOAD else 1
                sc_d[buf] = s_start(slot, j % GROUPS_PER_LOAD, buf)
            return carry

        lax.fori_loop(0, DBL, body, 0)
        g_wait_recon(0)
        s_wait_recon(1)
        plsc.subcore_barrier()

        pltpu.sync_copy(acc.at[pl.ds(r0, ROWS_PER_TILE)],
                        out_hbm.at[c, pl.ds(r0, ROWS_PER_TILE)])

        @pl.when(s == 0)
        def _():
            pltpu.sync_copy(
                acc.at[pl.ds(NS * ROWS_PER_TILE, TAIL_ROWS)],
                out_hbm.at[c, pl.ds(NS * ROWS_PER_TILE, TAIL_ROWS)])

    return k(xv, idx2, dstg, x)


def _mlp_kern(agg_ref, w1_ref, b1_ref, w2_ref, b2_ref, out_ref, *, leaky):
    a = agg_ref[...]
    h0 = jnp.concatenate([a[0], a[1]], axis=1)
    h = jnp.maximum(
        jnp.dot(h0, w1_ref[...], preferred_element_type=jnp.float32)
        + b1_ref[...], 0.0)
    o = (jnp.dot(h, w2_ref[...], preferred_element_type=jnp.float32)
         + b2_ref[...])
    if leaky:
        o = jnp.where(o >= 0, o, 0.01 * o)
    out_ref[...] = o


def _tc_mlp(agg, w1, b1, w2, b2, leaky):
    BN = 1000
    grid = (N // BN,)
    return pl.pallas_call(
        functools.partial(_mlp_kern, leaky=leaky),
        grid=grid,
        in_specs=[
            pl.BlockSpec((NC, BN, HALF), lambda i: (0, i, 0)),
            pl.BlockSpec((D, D), lambda i: (0, 0)),
            pl.BlockSpec((1, D), lambda i: (0, 0)),
            pl.BlockSpec((D, D), lambda i: (0, 0)),
            pl.BlockSpec((1, D), lambda i: (0, 0)),
        ],
        out_specs=pl.BlockSpec((BN, D), lambda i: (i, 0)),
        out_shape=jax.ShapeDtypeStruct((N, D), jnp.float32),
    )(agg, w1, b1.reshape(1, D), w2, b2.reshape(1, D))


def kernel(x, edge_index, W1_0, b1_0, W2_0, b2_0, W1_1, b1_1, W2_1, b2_1,
           W1_2, b1_2, W2_2, b2_2):
    src = edge_index[0].astype(jnp.int32)
    dst = edge_index[1].astype(jnp.int32)
    pad = NGRP_PAD * LANES - E
    src_p = jnp.concatenate([src, jnp.zeros((pad,), jnp.int32)])
    dst_p = jnp.concatenate([dst, jnp.full((pad,), TRASH, jnp.int32)])
    idx2 = jnp.stack([src_p * 2, src_p * 2 + 1]).reshape(NC, NGRP_PAD, LANES)
    dstg = dst_p.reshape(NGRP_PAD, LANES)

    h = x
    for (w1, b1, w2, b2, leaky) in (
            (W1_0, b1_0, W2_0, b2_0, True),
            (W1_1, b1_1, W2_1, b2_1, True),
            (W1_2, b1_2, W2_2, b2_2, False)):
        xplus = _sc_aggregate(h.reshape(2 * N, HALF), idx2, dstg, h)
        h = _tc_mlp(xplus, w1, b1, w2, b2, leaky)
    return h


# speedup vs baseline: 2.8892x; 1.0006x over previous
import functools

import jax
import jax.numpy as jnp
from jax import lax
from jax.experimental import pallas as pl
from jax.experimental.pallas import tpu as pltpu
from jax.experimental.pallas import tpu_sc as plsc

N = 10000
E = 160000
D = 256
HALF = D // 2

NC = 2
NS = 16
GATHER_SPLIT = 4
LANES = 128
GROUPS_PER_LOAD = 8

E_PAD = ((E + NS * GROUPS_PER_LOAD * LANES - 1)
         // (NS * GROUPS_PER_LOAD * LANES)) * (NS * GROUPS_PER_LOAD * LANES)
NGRP = E_PAD // LANES
NGRP_PAD = NGRP + GROUPS_PER_LOAD
GRP_PER_TILE = NGRP // NS
LOADS_PER_TILE = GRP_PER_TILE // GROUPS_PER_LOAD
DBL = LOADS_PER_TILE // 2

TRASH = N
ACC_ROWS = N + 8
ROWS_PER_TILE = 624
TAIL_ROWS = N - NS * ROWS_PER_TILE


def _sc_aggregate(xv, idx2, dstg, x):
    mesh = plsc.VectorSubcoreMesh(
        core_axis_name="c", subcore_axis_name="s", num_cores=NC,
        num_subcores=NS)

    @functools.partial(
        pl.kernel,
        mesh=mesh,
        out_type=jax.ShapeDtypeStruct((NC, N, HALF), jnp.float32),
        scratch_types=[
            pltpu.VMEM((2, 2, GROUPS_PER_LOAD, LANES), jnp.int32),
            pltpu.VMEM((2, LANES, HALF), jnp.float32),
            pltpu.VMEM_SHARED((ACC_ROWS, HALF), jnp.float32),
            pltpu.SemaphoreType.DMA,
            pltpu.SemaphoreType.DMA,
            pltpu.SemaphoreType.DMA,
            pltpu.SemaphoreType.DMA,
            pltpu.SemaphoreType.DMA,
        ],
    )
    def k(xv_hbm, idx2_hbm, dstg_hbm, x_hbm, out_hbm,
          idx_v, rows_v, acc, sg0, sg1, ss0, ss1, si):
        c = lax.axis_index("c")
        s = lax.axis_index("s")
        sem_g = (sg0, sg1)
        sem_s = (ss0, ss1)

        QS = LANES // GATHER_SPLIT

        def g_start(slot, grp, buf):
            for q in range(GATHER_SPLIT):
                pltpu.async_copy(
                    xv_hbm.at[idx_v.at[slot, 0, grp, pl.ds(q * QS, QS)]],
                    rows_v.at[buf, pl.ds(q * QS, QS)], sem_g[buf])
            return None

        def s_start(slot, grp, buf):
            return pltpu.async_copy(rows_v.at[buf],
                                    acc.at[idx_v.at[slot, 1, grp]],
                                    sem_s[buf], add=True)

        def g_wait_recon(buf):
            for q in range(GATHER_SPLIT):
                pltpu.make_async_copy(
                    xv_hbm.at[pl.ds(0, QS)],
                    rows_v.at[buf, pl.ds(0, QS)], sem_g[buf]).wait()

        def s_wait_recon(buf):
            pltpu.make_async_copy(xv_hbm.at[pl.ds(0, LANES)],
                                  rows_v.at[buf], sem_s[buf]).wait()

        def idx_load(slot, blk):
            g0 = s * GRP_PER_TILE + blk * GROUPS_PER_LOAD
            d1 = pltpu.async_copy(idx2_hbm.at[c, pl.ds(g0, GROUPS_PER_LOAD)],
                                  idx_v.at[slot, 0], si)
            d2 = pltpu.async_copy(dstg_hbm.at[pl.ds(g0, GROUPS_PER_LOAD)],
                                  idx_v.at[slot, 1], si)
            return d1, d2

        r0 = s * ROWS_PER_TILE
        pltpu.sync_copy(
            x_hbm.at[pl.ds(r0, ROWS_PER_TILE), pl.ds(c * HALF, HALF)],
            acc.at[pl.ds(r0, ROWS_PER_TILE)])

        @pl.when(s == 0)
        def _():
            pltpu.sync_copy(
                x_hbm.at[pl.ds(NS * ROWS_PER_TILE, TAIL_ROWS),
                         pl.ds(c * HALF, HALF)],
                acc.at[pl.ds(NS * ROWS_PER_TILE, TAIL_ROWS)])

        plsc.subcore_barrier()

        p1, p2 = idx_load(0, 0)
        p1.wait()
        p2.wait()
        g_start(0, 0, 0)

        def body(t, carry):
            sc_d = [None, None]
            g_d = [None, None]
            idx_d = None
            for j in range(2 * GROUPS_PER_LOAD):
                buf = j % 2
                nbuf = (j + 1) % 2
                if j == 0:
                    @pl.when(t > 0)
                    def _():
                        s_wait_recon(1)
                else:
                    sc_d[nbuf].wait()
                if j == 0:
                    idx_d = idx_load(1, 2 * t + 1)
                if j == GROUPS_PER_LOAD:
                    idx_d = idx_load(0, 2 * t + 2)
                if j == GROUPS_PER_LOAD - 1 or j == 2 * GROUPS_PER_LOAD - 1:
                    idx_d[0].wait()
                    idx_d[1].wait()
                if j < GROUPS_PER_LOAD - 1:
                    ns, ng = 0, j + 1
                elif j < 2 * GROUPS_PER_LOAD - 1:
                    ns, ng = 1, (j + 1) % GROUPS_PER_LOAD
                else:
                    ns, ng = 0, 0
                g_start(ns, ng, nbuf)
                g_wait_recon(buf)
                slot = 0 if j < GROUPS_PER_LOAD else 1
                sc_d[buf] = s_start(slot, j % GROUPS_PER_LOAD, buf)
            return carry

        lax.fori_loop(0, DBL, body, 0)
        g_wait_recon(0)
        s_wait_recon(1)
        plsc.subcore_barrier()

        pltpu.sync_copy(acc.at[pl.ds(r0, ROWS_PER_TILE)],
                        out_hbm.at[c, pl.ds(r0, ROWS_PER_TILE)])

        @pl.when(s == 0)
        def _():
            pltpu.sync_copy(
                acc.at[pl.ds(NS * ROWS_PER_TILE, TAIL_ROWS)],
                out_hbm.at[c, pl.ds(NS * ROWS_PER_TILE, TAIL_ROWS)])

    return k(xv, idx2, dstg, x)


def _mlp_kern(agg_ref, w1_ref, b1_ref, w2_ref, b2_ref, out_ref, *, leaky):
    a = agg_ref[...]
    h0 = jnp.concatenate([a[0], a[1]], axis=1)
    h = jnp.maximum(
        jnp.dot(h0, w1_ref[...], preferred_element_type=jnp.float32)
        + b1_ref[...], 0.0)
    o = (jnp.dot(h, w2_ref[...], preferred_element_type=jnp.float32)
         + b2_ref[...])
    if leaky:
        o = jnp.where(o >= 0, o, 0.01 * o)
    out_ref[...] = o


def _tc_mlp(agg, w1, b1, w2, b2, leaky):
    BN = 1000
    grid = (N // BN,)
    return pl.pallas_call(
        functools.partial(_mlp_kern, leaky=leaky),
        grid=grid,
        in_specs=[
            pl.BlockSpec((NC, BN, HALF), lambda i: (0, i, 0)),
            pl.BlockSpec((D, D), lambda i: (0, 0)),
            pl.BlockSpec((1, D), lambda i: (0, 0)),
            pl.BlockSpec((D, D), lambda i: (0, 0)),
            pl.BlockSpec((1, D), lambda i: (0, 0)),
        ],
        out_specs=pl.BlockSpec((BN, D), lambda i: (i, 0)),
        out_shape=jax.ShapeDtypeStruct((N, D), jnp.float32),
    )(agg, w1, b1.reshape(1, D), w2, b2.reshape(1, D))


def kernel(x, edge_index, W1_0, b1_0, W2_0, b2_0, W1_1, b1_1, W2_1, b2_1,
           W1_2, b1_2, W2_2, b2_2):
    src = edge_index[0].astype(jnp.int32)
    dst = edge_index[1].astype(jnp.int32)
    pad = NGRP_PAD * LANES - E
    src_p = jnp.concatenate([src, jnp.zeros((pad,), jnp.int32)])
    dst_p = jnp.concatenate([dst, jnp.full((pad,), TRASH, jnp.int32)])
    idx2 = jnp.stack([src_p * 2, src_p * 2 + 1]).reshape(NC, NGRP_PAD, LANES)
    dstg = dst_p.reshape(NGRP_PAD, LANES)

    h = x
    for (w1, b1, w2, b2, leaky) in (
            (W1_0, b1_0, W2_0, b2_0, True),
            (W1_1, b1_1, W2_1, b2_1, True),
            (W1_2, b1_2, W2_2, b2_2, False)):
        xplus = _sc_aggregate(h.reshape(2 * N, HALF), idx2, dstg, h)
        h = _tc_mlp(xplus, w1, b1, w2, b2, leaky)
    return h


# speedup vs baseline: 3.1738x; 1.0985x over previous
import functools

import jax
import jax.numpy as jnp
from jax import lax
from jax.experimental import pallas as pl
from jax.experimental.pallas import tpu as pltpu
from jax.experimental.pallas import tpu_sc as plsc

N = 10000
E = 160000
D = 256
HALF = D // 2

NC = 2
NS = 16
GATHER_SPLIT = 4
LANES = 128
GROUPS_PER_LOAD = 8

E_PAD = ((E + NS * GROUPS_PER_LOAD * LANES - 1)
         // (NS * GROUPS_PER_LOAD * LANES)) * (NS * GROUPS_PER_LOAD * LANES)
NGRP = E_PAD // LANES
NGRP_PAD = NGRP + GROUPS_PER_LOAD
GRP_PER_TILE = NGRP // NS
LOADS_PER_TILE = GRP_PER_TILE // GROUPS_PER_LOAD
DBL = LOADS_PER_TILE // 2

TRASH = N
ACC_ROWS = N + 8
ROWS_PER_TILE = 624
TAIL_ROWS = N - NS * ROWS_PER_TILE


def _sc_aggregate(xv, idx2, dstg):
    mesh = plsc.VectorSubcoreMesh(
        core_axis_name="c", subcore_axis_name="s", num_cores=NC,
        num_subcores=NS)

    @functools.partial(
        pl.kernel,
        mesh=mesh,
        out_type=jax.ShapeDtypeStruct((NC, N, HALF), jnp.float32),
        scratch_types=[
            pltpu.VMEM((2, 2, GROUPS_PER_LOAD, LANES), jnp.int32),
            pltpu.VMEM((2, LANES, HALF), jnp.float32),
            pltpu.VMEM_SHARED((ACC_ROWS, HALF), jnp.float32),
            pltpu.SemaphoreType.DMA,
            pltpu.SemaphoreType.DMA,
            pltpu.SemaphoreType.DMA,
            pltpu.SemaphoreType.DMA,
            pltpu.SemaphoreType.DMA,
        ],
    )
    def k(xv_hbm, idx2_hbm, dstg_hbm, out_hbm,
          idx_v, rows_v, acc, sg0, sg1, ss0, ss1, si):
        c = lax.axis_index("c")
        s = lax.axis_index("s")
        sem_g = (sg0, sg1)
        sem_s = (ss0, ss1)

        QS = LANES // GATHER_SPLIT

        def g_start(slot, grp, buf):
            for q in range(GATHER_SPLIT):
                pltpu.async_copy(
                    xv_hbm.at[idx_v.at[slot, 0, grp, pl.ds(q * QS, QS)]],
                    rows_v.at[buf, pl.ds(q * QS, QS)], sem_g[buf])
            return None

        def s_start(slot, grp, buf):
            return pltpu.async_copy(rows_v.at[buf],
                                    acc.at[idx_v.at[slot, 1, grp]],
                                    sem_s[buf], add=True)

        def g_wait_recon(buf):
            for q in range(GATHER_SPLIT):
                pltpu.make_async_copy(
                    xv_hbm.at[pl.ds(0, QS)],
                    rows_v.at[buf, pl.ds(0, QS)], sem_g[buf]).wait()

        def s_wait_recon(buf):
            pltpu.make_async_copy(xv_hbm.at[pl.ds(0, LANES)],
                                  rows_v.at[buf], sem_s[buf]).wait()

        def idx_load(slot, blk):
            g0 = s * GRP_PER_TILE + blk * GROUPS_PER_LOAD
            d1 = pltpu.async_copy(idx2_hbm.at[c, pl.ds(g0, GROUPS_PER_LOAD)],
                                  idx_v.at[slot, 0], si)
            d2 = pltpu.async_copy(dstg_hbm.at[c, pl.ds(g0, GROUPS_PER_LOAD)],
                                  idx_v.at[slot, 1], si)
            return d1, d2

        r0 = s * ROWS_PER_TILE
        pltpu.sync_copy(xv_hbm.at[pl.ds(c * N + r0, ROWS_PER_TILE)],
                        acc.at[pl.ds(r0, ROWS_PER_TILE)])

        @pl.when(s == 0)
        def _():
            pltpu.sync_copy(
                xv_hbm.at[pl.ds(c * N + NS * ROWS_PER_TILE, TAIL_ROWS)],
                acc.at[pl.ds(NS * ROWS_PER_TILE, TAIL_ROWS)])

        plsc.subcore_barrier()

        p1, p2 = idx_load(0, 0)
        p1.wait()
        p2.wait()
        g_start(0, 0, 0)

        def body(t, carry):
            sc_d = [None, None]
            g_d = [None, None]
            idx_d = None
            for j in range(2 * GROUPS_PER_LOAD):
                buf = j % 2
                nbuf = (j + 1) % 2
                if j == 0:
                    @pl.when(t > 0)
                    def _():
                        s_wait_recon(1)
                else:
                    sc_d[nbuf].wait()
                if j == 0:
                    idx_d = idx_load(1, 2 * t + 1)
                if j == GROUPS_PER_LOAD:
                    idx_d = idx_load(0, 2 * t + 2)
                if j == GROUPS_PER_LOAD - 1 or j == 2 * GROUPS_PER_LOAD - 1:
                    idx_d[0].wait()
                    idx_d[1].wait()
                if j < GROUPS_PER_LOAD - 1:
                    ns, ng = 0, j + 1
                elif j < 2 * GROUPS_PER_LOAD - 1:
                    ns, ng = 1, (j + 1) % GROUPS_PER_LOAD
                else:
                    ns, ng = 0, 0
                g_start(ns, ng, nbuf)
                g_wait_recon(buf)
                slot = 0 if j < GROUPS_PER_LOAD else 1
                sc_d[buf] = s_start(slot, j % GROUPS_PER_LOAD, buf)
            return carry

        lax.fori_loop(0, DBL, body, 0)
        g_wait_recon(0)
        s_wait_recon(1)
        plsc.subcore_barrier()

        pltpu.sync_copy(acc.at[pl.ds(r0, ROWS_PER_TILE)],
                        out_hbm.at[c, pl.ds(r0, ROWS_PER_TILE)])

        @pl.when(s == 0)
        def _():
            pltpu.sync_copy(
                acc.at[pl.ds(NS * ROWS_PER_TILE, TAIL_ROWS)],
                out_hbm.at[c, pl.ds(NS * ROWS_PER_TILE, TAIL_ROWS)])

    return k(xv, idx2, dstg)


def _mlp_kern(agg_ref, w1_ref, b1_ref, w2_ref, b2_ref, out_ref, *, leaky,
              split_out):
    a = agg_ref[...]
    h0 = jnp.concatenate([a[0], a[1]], axis=1)
    h = jnp.maximum(
        jnp.dot(h0, w1_ref[...], preferred_element_type=jnp.float32)
        + b1_ref[...], 0.0)
    o = (jnp.dot(h, w2_ref[...], preferred_element_type=jnp.float32)
         + b2_ref[...])
    if leaky:
        o = jnp.where(o >= 0, o, 0.01 * o)
    if split_out:
        out_ref[0] = o[:, :HALF]
        out_ref[1] = o[:, HALF:]
    else:
        out_ref[...] = o


def _tc_mlp(agg, w1, b1, w2, b2, leaky, split_out):
    BN = 1000
    grid = (N // BN,)
    if split_out:
        out_spec = pl.BlockSpec((NC, BN, HALF), lambda i: (0, i, 0))
        out_shape = jax.ShapeDtypeStruct((NC, N, HALF), jnp.float32)
    else:
        out_spec = pl.BlockSpec((BN, D), lambda i: (i, 0))
        out_shape = jax.ShapeDtypeStruct((N, D), jnp.float32)
    return pl.pallas_call(
        functools.partial(_mlp_kern, leaky=leaky, split_out=split_out),
        grid=grid,
        in_specs=[
            pl.BlockSpec((NC, BN, HALF), lambda i: (0, i, 0)),
            pl.BlockSpec((D, D), lambda i: (0, 0)),
            pl.BlockSpec((1, D), lambda i: (0, 0)),
            pl.BlockSpec((D, D), lambda i: (0, 0)),
            pl.BlockSpec((1, D), lambda i: (0, 0)),
        ],
        out_specs=out_spec,
        out_shape=out_shape,
    )(agg, w1, b1.reshape(1, D), w2, b2.reshape(1, D))


def kernel(x, edge_index, W1_0, b1_0, W2_0, b2_0, W1_1, b1_1, W2_1, b2_1,
           W1_2, b1_2, W2_2, b2_2):
    src = edge_index[0].astype(jnp.int32)
    dst = edge_index[1].astype(jnp.int32)
    pad = NGRP_PAD * LANES - E
    src_p = jnp.concatenate([src, jnp.zeros((pad,), jnp.int32)])
    dst_p = jnp.concatenate([dst, jnp.full((pad,), TRASH, jnp.int32)])
    idx2 = jnp.stack([src_p, src_p + N]).reshape(NC, NGRP_PAD, LANES)
    dstg = jnp.stack([dst_p, dst_p]).reshape(NC, NGRP_PAD, LANES)

    hs = jnp.moveaxis(x.reshape(N, NC, HALF), 1, 0)
    for (w1, b1, w2, b2) in ((W1_0, b1_0, W2_0, b2_0),
                             (W1_1, b1_1, W2_1, b2_1)):
        xplus = _sc_aggregate(hs.reshape(NC * N, HALF), idx2, dstg)
        hs = _tc_mlp(xplus, w1, b1, w2, b2, leaky=True, split_out=True)
    xplus = _sc_aggregate(hs.reshape(NC * N, HALF), idx2, dstg)
    return _tc_mlp(xplus, W1_2, b1_2, W2_2, b2_2, leaky=False,
                   split_out=False)
